# baseline (speedup 1.0000x reference)
import jax
import jax.numpy as jnp
from jax.experimental import pallas as pl
from jax.experimental.pallas import tpu as pltpu

IN_CH = 3
OUT_CH = 128
BATCH = 256
TOTAL = 2048
TOPK = 50
EPS = 1e-5
NCHUNK = TOTAL // BATCH


def _bn_kernel(x_ref, w_ref, b_ref, g_ref, be_ref, o_ref):
    out = jax.lax.dot_general(x_ref[0], w_ref[...], (((1,), (1,)), ((), ())),
                              preferred_element_type=jnp.float32)
    out = out + b_ref[...]
    mu = jnp.mean(out, axis=0, keepdims=True)
    var = jnp.mean((out - mu) ** 2, axis=0, keepdims=True)
    o_ref[0] = (out - mu) / jnp.sqrt(var + EPS) * g_ref[...] + be_ref[...]


def _score_kernel(xall_ref, w8_ref, bs_ref, s_ref):
    blk = pl.program_id(0)
    X = xall_ref[...]
    X2 = X * X
    W8 = w8_ref[...]

    def row_body(i, carry):
        xi = xall_ref[pl.ds(blk * BATCH + i, 1), :]
        d2 = (xi * xi + X2) - 2.0 * (xi * X)
        res = jax.lax.dot_general(d2, W8, (((1,), (0,)), ((), ())),
                                  preferred_element_type=jnp.float32)
        row = jnp.transpose(res)[0:1, :]
        s_ref[pl.ds(i, 1), :] = row
        return carry

    jax.lax.fori_loop(0, BATCH, row_body, 0)

    scores = jnp.maximum(s_ref[...] + bs_ref[0, 0], 0.0)
    bits = jax.lax.bitcast_convert_type(scores, jnp.int32)

    def body(_, carry):
        lo, hi = carry
        mid = lo + ((hi - lo) >> 1)
        cnt = jnp.sum(jnp.where(bits >= mid, 1, 0), axis=1, keepdims=True)
        ok = cnt >= TOPK
        return jnp.where(ok, mid, lo), jnp.where(ok, hi, mid)

    lo0 = jnp.zeros((BATCH, 1), jnp.int32)
    hi0 = jnp.full((BATCH, 1), 0x7F800000, jnp.int32)
    lo, _ = jax.lax.fori_loop(0, 31, body, (lo0, hi0))

    n_gt = jnp.sum(jnp.where(bits > lo, 1, 0), axis=1, keepdims=True)
    need = TOPK - n_gt
    eq = bits == lo
    c = jnp.where(eq, 1, 0)
    p = c
    sh = 1
    while sh < TOTAL:
        p = p + jnp.concatenate(
            [jnp.zeros((BATCH, sh), jnp.int32), p[:, :TOTAL - sh]], axis=1)
        sh *= 2
    sel = eq & ((p - c) < need)

    mask = (bits > lo) | sel
    m = jnp.max(scores, axis=1, keepdims=True)
    e = jnp.where(mask, jnp.exp(scores - m), 0.0)
    z = jnp.sum(e, axis=1, keepdims=True)
    s_ref[...] = e / z


def kernel(inputs, W_lin, b_lin, gamma, beta, W_s, b_s):
    flat = inputs.reshape(NCHUNK, BATCH, -1)
    outputs = pl.pallas_call(
        _bn_kernel,
        grid=(NCHUNK,),
        in_specs=[
            pl.BlockSpec((1, BATCH, flat.shape[-1]), lambda i: (i, 0, 0)),
            pl.BlockSpec((OUT_CH, flat.shape[-1]), lambda i: (0, 0)),
            pl.BlockSpec((1, OUT_CH), lambda i: (0, 0)),
            pl.BlockSpec((1, OUT_CH), lambda i: (0, 0)),
            pl.BlockSpec((1, OUT_CH), lambda i: (0, 0)),
        ],
        out_specs=pl.BlockSpec((1, BATCH, OUT_CH), lambda i: (i, 0, 0)),
        out_shape=jax.ShapeDtypeStruct((NCHUNK, BATCH, OUT_CH), jnp.float32),
    )(flat, W_lin, b_lin[None, :], gamma[None, :], beta[None, :])
    outputs = outputs.reshape(TOTAL, OUT_CH)

    W8 = jnp.zeros((OUT_CH, OUT_CH), jnp.float32).at[:, 0].set(W_s[0])

    S = pl.pallas_call(
        _score_kernel,
        grid=(NCHUNK,),
        in_specs=[
            pl.BlockSpec((TOTAL, OUT_CH), lambda i: (0, 0)),
            pl.BlockSpec((OUT_CH, OUT_CH), lambda i: (0, 0)),
            pl.BlockSpec(memory_space=pltpu.SMEM),
        ],
        out_specs=pl.BlockSpec((BATCH, TOTAL), lambda i: (i, 0)),
        out_shape=jax.ShapeDtypeStruct((TOTAL, TOTAL), jnp.float32),
    )(outputs, W8, b_s.reshape(1, 1))
    return outputs, S


# speedup vs baseline: 2.0862x; 2.0862x over previous
import jax
import jax.numpy as jnp
from jax.experimental import pallas as pl
from jax.experimental.pallas import tpu as pltpu

IN_CH = 3
OUT_CH = 128
BATCH = 256
TOTAL = 2048
TOPK = 50
EPS = 1e-5
NCHUNK = TOTAL // BATCH


def _bn_kernel(x_ref, w_ref, b_ref, g_ref, be_ref, o_ref):
    out = jax.lax.dot_general(x_ref[0], w_ref[...], (((1,), (1,)), ((), ())),
                              preferred_element_type=jnp.float32)
    out = out + b_ref[...]
    mu = jnp.mean(out, axis=0, keepdims=True)
    var = jnp.mean((out - mu) ** 2, axis=0, keepdims=True)
    o_ref[0] = (out - mu) / jnp.sqrt(var + EPS) * g_ref[...] + be_ref[...]


def _score_kernel(xall_ref, wbd_ref, bs_ref, s_ref):
    blk = pl.program_id(0)
    X = xall_ref[...]
    X2 = X * X

    for g in range(BATCH // OUT_CH):
        res = jnp.zeros((TOTAL, OUT_CH), jnp.float32)
        for c in range(OUT_CH // 8):
            parts = []
            for jj in range(8):
                r = g * OUT_CH + c * 8 + jj
                xi = xall_ref[pl.ds(blk * BATCH + r, 1), :]
                parts.append((xi * xi + X2) - 2.0 * (xi * X))
            lhs = jnp.concatenate(parts, axis=1)
            res = res + jax.lax.dot_general(
                lhs, wbd_ref[c], (((1,), (0,)), ((), ())),
                preferred_element_type=jnp.float32)
        s_ref[g * OUT_CH:(g + 1) * OUT_CH, :] = jnp.transpose(res)

    scores = jnp.maximum(s_ref[...] + bs_ref[0, 0], 0.0)
    bits = jax.lax.bitcast_convert_type(scores, jnp.int32)

    def body(_, carry):
        lo, hi = carry
        mid = lo + ((hi - lo) >> 1)
        cnt = jnp.sum(jnp.where(bits >= mid, 1, 0), axis=1, keepdims=True)
        ok = cnt >= TOPK
        return jnp.where(ok, mid, lo), jnp.where(ok, hi, mid)

    lo0 = jnp.zeros((BATCH, 1), jnp.int32)
    hi0 = jnp.full((BATCH, 1), 0x7F800000, jnp.int32)
    lo, _ = jax.lax.fori_loop(0, 31, body, (lo0, hi0))

    n_gt = jnp.sum(jnp.where(bits > lo, 1, 0), axis=1, keepdims=True)
    need = TOPK - n_gt
    eq = bits == lo
    c = jnp.where(eq, 1, 0)
    p = c
    sh = 1
    while sh < TOTAL:
        p = p + jnp.concatenate(
            [jnp.zeros((BATCH, sh), jnp.int32), p[:, :TOTAL - sh]], axis=1)
        sh *= 2
    sel = eq & ((p - c) < need)

    mask = (bits > lo) | sel
    m = jnp.max(scores, axis=1, keepdims=True)
    e = jnp.where(mask, jnp.exp(scores - m), 0.0)
    z = jnp.sum(e, axis=1, keepdims=True)
    s_ref[...] = e / z


def kernel(inputs, W_lin, b_lin, gamma, beta, W_s, b_s):
    flat = inputs.reshape(NCHUNK, BATCH, -1)
    outputs = pl.pallas_call(
        _bn_kernel,
        grid=(NCHUNK,),
        in_specs=[
            pl.BlockSpec((1, BATCH, flat.shape[-1]), lambda i: (i, 0, 0)),
            pl.BlockSpec((OUT_CH, flat.shape[-1]), lambda i: (0, 0)),
            pl.BlockSpec((1, OUT_CH), lambda i: (0, 0)),
            pl.BlockSpec((1, OUT_CH), lambda i: (0, 0)),
            pl.BlockSpec((1, OUT_CH), lambda i: (0, 0)),
        ],
        out_specs=pl.BlockSpec((1, BATCH, OUT_CH), lambda i: (i, 0, 0)),
        out_shape=jax.ShapeDtypeStruct((NCHUNK, BATCH, OUT_CH), jnp.float32),
    )(flat, W_lin, b_lin[None, :], gamma[None, :], beta[None, :])
    outputs = outputs.reshape(TOTAL, OUT_CH)

    nch = OUT_CH // 8
    kk = jnp.arange(8 * OUT_CH)
    col = jnp.arange(OUT_CH)
    cc = jnp.arange(nch)
    Wbd = jnp.where(
        (cc[:, None, None] * 8 + (kk[None, :, None] // OUT_CH)) == col[None, None, :],
        W_s[0][kk % OUT_CH][None, :, None], 0.0).astype(jnp.float32)

    S = pl.pallas_call(
        _score_kernel,
        grid=(NCHUNK,),
        in_specs=[
            pl.BlockSpec((TOTAL, OUT_CH), lambda i: (0, 0)),
            pl.BlockSpec((nch, 8 * OUT_CH, OUT_CH), lambda i: (0, 0, 0)),
            pl.BlockSpec(memory_space=pltpu.SMEM),
        ],
        out_specs=pl.BlockSpec((BATCH, TOTAL), lambda i: (i, 0)),
        out_shape=jax.ShapeDtypeStruct((TOTAL, TOTAL), jnp.float32),
    )(outputs, Wbd, b_s.reshape(1, 1))
    return outputs, S
